# Initial kernel scaffold; baseline (speedup 1.0000x reference)
import functools

import jax
import jax.numpy as jnp
from jax import lax
from jax.experimental import pallas as pl
from jax.experimental.pallas import tpu as pltpu
from jax.experimental.pallas import tpu_sc as plsc


def _make_sc_embed(V, D, B, L, NC, NS):
    NW = NC * NS
    cols_per_tile = B // NW
    pairs_per_tile = cols_per_tile * L
    chunk = 64
    n_chunks = pairs_per_tile // chunk
    rows_per_sc = B // NC

    mesh = plsc.VectorSubcoreMesh(core_axis_name="c", subcore_axis_name="s")

    @functools.partial(
        pl.kernel,
        mesh=mesh,
        out_type=(
            jax.ShapeDtypeStruct((B, D), jnp.float32),
            jax.ShapeDtypeStruct((B, D), jnp.float32),
        ),
        scratch_types=[
            pltpu.VMEM((n_chunks, chunk), jnp.int32),
            pltpu.VMEM((n_chunks, chunk), jnp.int32),
            pltpu.VMEM((chunk, D), jnp.float32),
            pltpu.VMEM((chunk, D), jnp.float32),
            pltpu.VMEM_SHARED((rows_per_sc, D), jnp.float32),
            pltpu.VMEM_SHARED((rows_per_sc, D), jnp.float32),
            pltpu.SemaphoreType.DMA,
            pltpu.SemaphoreType.DMA,
        ],
    )
    def sc_embed(idx_hbm, dst_hbm, zeros_hbm, lut_hbm, slut_hbm,
                 e1_hbm, e2_hbm,
                 idx_v, dst_v, buf1, buf2, acc1, acc2, sem1, sem2):
        c = lax.axis_index("c")
        s = lax.axis_index("s")
        w = c * NS + s
        col_base = c * rows_per_sc + s * cols_per_tile
        loc_base = s * cols_per_tile

        pltpu.sync_copy(idx_hbm.at[w], idx_v)
        pltpu.sync_copy(dst_hbm.at[s], dst_v)
        pltpu.sync_copy(zeros_hbm, acc1.at[pl.ds(loc_base, cols_per_tile)])
        pltpu.sync_copy(zeros_hbm, acc2.at[pl.ds(loc_base, cols_per_tile)])

        def body(g, carry):
            cp1 = pltpu.async_copy(lut_hbm.at[idx_v.at[g]], buf1, sem1)
            cp2 = pltpu.async_copy(slut_hbm.at[idx_v.at[g]], buf2, sem2)
            cp1.wait()
            cp2.wait()
            pltpu.sync_copy(buf1, acc1.at[dst_v.at[g]], add=True)
            pltpu.sync_copy(buf2, acc2.at[dst_v.at[g]], add=True)
            return carry

        lax.fori_loop(0, n_chunks, body, 0)

        pltpu.sync_copy(acc1.at[pl.ds(loc_base, cols_per_tile)],
                        e1_hbm.at[pl.ds(col_base, cols_per_tile)])
        pltpu.sync_copy(acc2.at[pl.ds(loc_base, cols_per_tile)],
                        e2_hbm.at[pl.ds(col_base, cols_per_tile)])

    return sc_embed, cols_per_tile, pairs_per_tile, n_chunks, chunk


def _mlp_body(D, e1_ref, e2_ref, w1_ref, b1_ref, w2_ref, b2_ref, out_ref):
    w1 = w1_ref[...]
    a = lax.dot_general(e1_ref[...], w1[:, :D], (((1,), (1,)), ((), ())),
                        preferred_element_type=jnp.float32)
    a = a + lax.dot_general(e2_ref[...], w1[:, D:], (((1,), (1,)), ((), ())),
                            preferred_element_type=jnp.float32)
    h = jnp.maximum(a + b1_ref[...][None, :], 0.0)
    out_ref[...] = jnp.sum(h * w2_ref[...], axis=1) + b2_ref[...]


def kernel(input, lut, static_lut, W1, b1, W2, b2):
    L, B = input.shape
    V, D = lut.shape
    info = plsc.get_sparse_core_info()
    NC, NS = info.num_cores, info.num_subcores
    NW = NC * NS

    sc_embed, cols_per_tile, pairs_per_tile, n_chunks, chunk = _make_sc_embed(
        V, D, B, L, NC, NS)

    idx = input.T.astype(jnp.int32).reshape(NW, n_chunks, chunk)
    dst = (jnp.arange(NS, dtype=jnp.int32)[:, None] * cols_per_tile
           + (jnp.arange(pairs_per_tile, dtype=jnp.int32) // L)[None, :])
    dst = dst.reshape(NS, n_chunks, chunk)
    zeros = jnp.zeros((cols_per_tile, D), jnp.float32)

    e1, e2 = sc_embed(idx, dst, zeros, lut, static_lut)

    out = pl.pallas_call(
        functools.partial(_mlp_body, D),
        out_shape=jax.ShapeDtypeStruct((B,), jnp.float32),
    )(e1, e2, W1, b1, W2, b2)
    return out


# speedup vs baseline: 1.7450x; 1.7450x over previous
import functools

import jax
import jax.numpy as jnp
from jax import lax
from jax.experimental import pallas as pl
from jax.experimental.pallas import tpu as pltpu
from jax.experimental.pallas import tpu_sc as plsc

_LANES = 128
_SUB = 5
_LK = 25


def _make_sc_embed(n_rows, B, L, NC, NS):
    NW = NC * NS
    cols_per_tile = B // NW
    pairs_per_tile = cols_per_tile * L
    n_chunks = pairs_per_tile // _LK
    n_body = n_chunks // 2
    acc_rows = (B // NC) * _SUB + NS
    out_rows = B * _SUB

    mesh = plsc.VectorSubcoreMesh(core_axis_name="c", subcore_axis_name="s")

    @functools.partial(
        pl.kernel,
        mesh=mesh,
        compiler_params=pltpu.CompilerParams(use_tc_tiling_on_sc=False),
        out_type=jax.ShapeDtypeStruct((out_rows, _LANES), jnp.float32),
        scratch_types=[
            pltpu.VMEM((n_chunks, _LANES), jnp.int32),
            pltpu.VMEM((n_chunks, _LANES), jnp.int32),
            pltpu.VMEM((_LANES, _LANES), jnp.float32),
            pltpu.VMEM((_LANES, _LANES), jnp.float32),
            pltpu.VMEM_SHARED((acc_rows, _LANES), jnp.float32),
            pltpu.SemaphoreType.DMA,
            pltpu.SemaphoreType.DMA,
        ],
    )
    def sc_embed(idx_hbm, dst_hbm, zeros_hbm, tab_hbm, ep_hbm,
                 idx_v, dst_v, buf_a, buf_b, acc, gs_a, gs_b):
        c = lax.axis_index("c")
        s = lax.axis_index("s")
        w = c * NS + s
        own = cols_per_tile * _SUB
        loc_base = s * own
        out_base = (c * (B // NC) + s * cols_per_tile) * _SUB

        pltpu.sync_copy(idx_hbm.at[w], idx_v)
        pltpu.sync_copy(dst_hbm.at[s], dst_v)
        pltpu.sync_copy(zeros_hbm, acc.at[pl.ds(loc_base, own)])
        pltpu.sync_copy(zeros_hbm.at[pl.ds(0, 1)],
                        acc.at[pl.ds((B // NC) * _SUB + s, 1)])

        pltpu.async_copy(tab_hbm.at[idx_v.at[0]], buf_a, gs_a)
        pltpu.async_copy(tab_hbm.at[idx_v.at[1]], buf_b, gs_b)

        def body(k, carry):
            ga = 2 * k
            gb = 2 * k + 1
            pltpu.make_async_copy(tab_hbm.at[idx_v.at[ga]], buf_a, gs_a).wait()
            pltpu.sync_copy(buf_a, acc.at[dst_v.at[ga]], add=True)

            @pl.when(k < n_body - 1)
            def _():
                pltpu.async_copy(tab_hbm.at[idx_v.at[ga + 2]], buf_a, gs_a)

            pltpu.make_async_copy(tab_hbm.at[idx_v.at[gb]], buf_b, gs_b).wait()
            pltpu.sync_copy(buf_b, acc.at[dst_v.at[gb]], add=True)

            @pl.when(k < n_body - 1)
            def _():
                pltpu.async_copy(tab_hbm.at[idx_v.at[gb + 2]], buf_b, gs_b)

            return carry

        lax.fori_loop(0, n_body, body, 0)

        pltpu.sync_copy(acc.at[pl.ds(loc_base, own)],
                        ep_hbm.at[pl.ds(out_base, own)])

    return sc_embed, cols_per_tile, pairs_per_tile, n_chunks


def _mlp_body(ep_ref, wx_ref, b1_ref, w2_ref, b2_ref, out_ref):
    a = lax.dot_general(ep_ref[...], wx_ref[...], (((1,), (0,)), ((), ())),
                        preferred_element_type=jnp.float32)
    h = jnp.maximum(a + b1_ref[...][None, :], 0.0)
    out_ref[...] = jnp.sum(h * w2_ref[...], axis=1) + b2_ref[...]


def kernel(input, lut, static_lut, W1, b1, W2, b2):
    L, B = input.shape
    V, D = lut.shape
    Dp = _SUB * _LANES
    info = plsc.get_sparse_core_info()
    NC, NS = info.num_cores, info.num_subcores
    NW = NC * NS

    sc_embed, cols_per_tile, pairs_per_tile, n_chunks = _make_sc_embed(
        V * _SUB, B, L, NC, NS)

    fused = jnp.concatenate(
        [lut, static_lut, jnp.zeros((V, Dp - 2 * D), jnp.float32)], axis=1)
    fused = fused.reshape(V * _SUB, _LANES)

    idx_lk = input.astype(jnp.int32).reshape(L, NW, cols_per_tile)
    idx_lk = idx_lk.transpose(1, 0, 2).reshape(NW, n_chunks, _LK)
    idx5 = (idx_lk[..., None] * _SUB
            + jnp.arange(_SUB, dtype=jnp.int32)).reshape(NW, n_chunks, _LK * _SUB)
    pad_src = jnp.broadcast_to(
        (jnp.arange(NW, dtype=jnp.int32) % NS)[:, None, None],
        (NW, n_chunks, _LANES - _LK * _SUB))
    idx = jnp.concatenate([idx5, pad_src], axis=2)

    p = jnp.arange(pairs_per_tile, dtype=jnp.int32)
    col = p % cols_per_tile
    dst1 = ((jnp.arange(NS, dtype=jnp.int32)[:, None] * cols_per_tile + col)
            * _SUB)
    dst5 = (dst1[..., None] + jnp.arange(_SUB, dtype=jnp.int32)).reshape(
        NS, n_chunks, _LK * _SUB)
    pad_dst = jnp.broadcast_to(
        ((B // NC) * _SUB + jnp.arange(NS, dtype=jnp.int32))[:, None, None],
        (NS, n_chunks, _LANES - _LK * _SUB))
    dst = jnp.concatenate([dst5, pad_dst], axis=2)

    zeros = jnp.zeros((cols_per_tile * _SUB, _LANES), jnp.float32)

    ep = sc_embed(idx, dst, zeros, fused)
    ep = ep.reshape(B, Dp)

    Wx = jnp.concatenate(
        [W1.T, jnp.zeros((Dp - 2 * D, 2 * D), jnp.float32)], axis=0)
    out = pl.pallas_call(
        _mlp_body,
        out_shape=jax.ShapeDtypeStruct((B,), jnp.float32),
    )(ep, Wx, b1, W2, b2)
    return out


# speedup vs baseline: 1.7606x; 1.0090x over previous
import functools

import jax
import jax.numpy as jnp
from jax import lax
from jax.experimental import pallas as pl
from jax.experimental.pallas import tpu as pltpu
from jax.experimental.pallas import tpu_sc as plsc

_LANES = 128
_SUB = 5
_LK = 25
_NBLK = _LANES // 16


def _make_sc_embed(B, L, NC, NS):
    NW = NC * NS
    cols_per_tile = B // NW
    pairs_per_tile = cols_per_tile * L
    n_chunks = pairs_per_tile // _LK
    cpc = L // _LK
    n_acc = _SUB * _NBLK
    out_rows = B * _SUB

    mesh = plsc.VectorSubcoreMesh(core_axis_name="c", subcore_axis_name="s")

    @functools.partial(
        pl.kernel,
        mesh=mesh,
        compiler_params=pltpu.CompilerParams(use_tc_tiling_on_sc=False),
        out_type=jax.ShapeDtypeStruct((out_rows, _LANES), jnp.float32),
        scratch_types=[
            pltpu.VMEM((n_chunks, _LANES), jnp.int32),
            pltpu.VMEM((_LANES, _LANES), jnp.float32),
            pltpu.VMEM((_LANES, _LANES), jnp.float32),
            pltpu.VMEM((cols_per_tile * _SUB, _LANES), jnp.float32),
            pltpu.SemaphoreType.DMA,
            pltpu.SemaphoreType.DMA,
        ],
    )
    def sc_embed(idx_hbm, tab_hbm, ep_hbm,
                 idx_v, buf_a, buf_b, stage, gs_a, gs_b):
        c = lax.axis_index("c")
        s = lax.axis_index("s")
        w = c * NS + s
        out_base = (c * (B // NC) + s * cols_per_tile) * _SUB

        pltpu.sync_copy(idx_hbm.at[w], idx_v)

        def accumulate(buf, acc):
            def lk_body(lk, a):
                a = list(a)
                for t in range(_SUB):
                    row = _SUB * lk + t
                    for blk in range(_NBLK):
                        a[t * _NBLK + blk] = (
                            a[t * _NBLK + blk]
                            + buf[row, pl.ds(blk * 16, 16)])
                return tuple(a)
            return lax.fori_loop(0, _LK, lk_body, acc)

        def col_body(col, carry):
            base = col * cpc
            pltpu.async_copy(tab_hbm.at[idx_v.at[base]], buf_a, gs_a)
            pltpu.async_copy(tab_hbm.at[idx_v.at[base + 1]], buf_b, gs_b)
            acc = tuple(jnp.zeros((16,), jnp.float32) for _ in range(n_acc))

            def pair_body(kp, a):
                ga = base + 2 * kp
                pltpu.make_async_copy(
                    tab_hbm.at[idx_v.at[ga]], buf_a, gs_a).wait()
                a = accumulate(buf_a, a)

                @pl.when(kp < cpc // 2 - 1)
                def _():
                    pltpu.async_copy(
                        tab_hbm.at[idx_v.at[ga + 2]], buf_a, gs_a)

                pltpu.make_async_copy(
                    tab_hbm.at[idx_v.at[ga + 1]], buf_b, gs_b).wait()
                a = accumulate(buf_b, a)

                @pl.when(kp < cpc // 2 - 1)
                def _():
                    pltpu.async_copy(
                        tab_hbm.at[idx_v.at[ga + 3]], buf_b, gs_b)

                return a

            acc = lax.fori_loop(0, cpc // 2, pair_body, acc)
            for t in range(_SUB):
                for blk in range(_NBLK):
                    stage[col * _SUB + t, pl.ds(blk * 16, 16)] = (
                        acc[t * _NBLK + blk])
            return carry

        lax.fori_loop(0, cols_per_tile, col_body, 0)

        pltpu.sync_copy(stage,
                        ep_hbm.at[pl.ds(out_base, cols_per_tile * _SUB)])

    return sc_embed, cols_per_tile, pairs_per_tile, n_chunks


def _mlp_body(ep_ref, wx_ref, b1_ref, w2_ref, b2_ref, out_ref):
    a = lax.dot_general(ep_ref[...], wx_ref[...], (((1,), (0,)), ((), ())),
                        preferred_element_type=jnp.float32)
    h = jnp.maximum(a + b1_ref[...][None, :], 0.0)
    out_ref[...] = jnp.sum(h * w2_ref[...], axis=1) + b2_ref[...]


def kernel(input, lut, static_lut, W1, b1, W2, b2):
    L, B = input.shape
    V, D = lut.shape
    Dp = _SUB * _LANES
    info = plsc.get_sparse_core_info()
    NC, NS = info.num_cores, info.num_subcores
    NW = NC * NS

    sc_embed, cols_per_tile, pairs_per_tile, n_chunks = _make_sc_embed(
        B, L, NC, NS)

    fused = (jnp.pad(lut, ((0, 0), (0, Dp - D)))
             + jnp.pad(static_lut, ((0, 0), (D, Dp - 2 * D))))
    fused = fused.reshape(V * _SUB, _LANES)

    idx_lk = input.T.astype(jnp.int32).reshape(NW, n_chunks, _LK)
    idx5 = (idx_lk[..., None] * _SUB
            + jnp.arange(_SUB, dtype=jnp.int32)).reshape(NW, n_chunks,
                                                         _LK * _SUB)
    pad_src = jnp.broadcast_to(
        (jnp.arange(NW, dtype=jnp.int32) % NS)[:, None, None],
        (NW, n_chunks, _LANES - _LK * _SUB))
    idx = jnp.concatenate([idx5, pad_src], axis=2)

    ep = sc_embed(idx, fused)
    ep = ep.reshape(B, Dp)

    Wx = jnp.concatenate(
        [W1.T, jnp.zeros((Dp - 2 * D, 2 * D), jnp.float32)], axis=0)
    out = pl.pallas_call(
        _mlp_body,
        out_shape=jax.ShapeDtypeStruct((B,), jnp.float32),
    )(ep, Wx, b1, W2, b2)
    return out
